# baseline (speedup 1.0000x reference)
import functools

import jax
import jax.numpy as jnp
from jax import lax
from jax.experimental import pallas as pl
from jax.experimental.pallas import tpu as pltpu
from jax.experimental.pallas import tpu_sc as plsc

NC, NS = 2, 16
NW = NC * NS
NBUF = 5


def _gather_body(h_hbm, ei_hbm, out_hbm, h_sp, idx_v, *bufs_and_sems,
                 ntask, ec, e_per_w, rows_per_tile):
    bufs = bufs_and_sems[:NBUF]
    sgs = bufs_and_sems[NBUF:2 * NBUF]
    sws = bufs_and_sems[2 * NBUF:]
    cid = lax.axis_index("c")
    sid = lax.axis_index("s")
    wid = cid * NS + sid
    e0 = wid * e_per_w
    d = h_sp.shape[1]
    n_edges = ei_hbm.shape[0] // 2
    n_rows = h_sp.shape[0]
    off = pl.multiple_of(jnp.minimum(sid * rows_per_tile, n_rows - rows_per_tile), 8)
    cp_h = pltpu.make_async_copy(h_hbm.at[pl.ds(off, rows_per_tile)],
                                 h_sp.at[pl.ds(off, rows_per_tile)], sgs[0])
    cp_s = pltpu.make_async_copy(ei_hbm.at[pl.ds(e0, e_per_w)],
                                 idx_v.at[pl.ds(0, e_per_w)], sgs[1])
    cp_d = pltpu.make_async_copy(ei_hbm.at[pl.ds(n_edges + e0, e_per_w)],
                                 idx_v.at[pl.ds(e_per_w, e_per_w)], sgs[2])
    cp_h.start()
    cp_s.start()
    cp_d.start()
    cp_h.wait()
    cp_s.wait()
    cp_d.wait()
    plsc.subcore_barrier()

    def gather(k, i):
        src = pl.multiple_of((k % 2) * e_per_w + (k // 2) * ec, 8)
        return pltpu.make_async_copy(h_sp.at[idx_v.at[pl.ds(src, ec)]],
                                     bufs[i], sgs[i])

    def writeback(k, i):
        rows = pl.ds(e0 + (k // 2) * ec, ec)
        col = pl.multiple_of((k % 2) * d, d)
        return pltpu.make_async_copy(bufs[i], out_hbm.at[rows, pl.ds(col, d)],
                                     sws[i])

    def slot(k, i):
        gather(k, i).wait()
        writeback(k, i).start()

        @pl.when(k > 0)
        def _():
            writeback(k - 1, (i - 1) % NBUF).wait()

        @pl.when(k + NBUF - 2 < ntask)
        def _():
            gather(k + NBUF - 2, (i + NBUF - 2) % NBUF).start()

    def group(t, carry):
        k0 = NBUF * t
        for i in range(NBUF):
            slot(k0 + i, i)
        return carry

    for i in range(NBUF - 2):
        gather(i, i).start()
    lax.fori_loop(0, ntask // NBUF, group, 0)
    writeback(ntask - 1, (ntask - 1) % NBUF).wait()


def kernel(h, edge_index):
    n, d = h.shape
    e = edge_index.shape[1]
    e_per_w = e // NW
    ec = 40
    ntask = 2 * (e_per_w // ec)
    assert ntask % NBUF == 0

    ei = edge_index.astype(jnp.int32).reshape(-1)
    rpt = ((n + NS - 1) // NS + 7) // 8 * 8
    body = functools.partial(_gather_body, ntask=ntask, ec=ec,
                             e_per_w=e_per_w, rows_per_tile=rpt)
    return pl.kernel(
        body,
        out_type=jax.ShapeDtypeStruct((e, 2 * d), jnp.float32),
        mesh=plsc.VectorSubcoreMesh(core_axis_name="c", subcore_axis_name="s"),
        scratch_types=(
            [pltpu.VMEM_SHARED((n, d), jnp.float32),
             pltpu.VMEM((2 * e_per_w,), jnp.int32)]
            + [pltpu.VMEM((ec, d), jnp.float32)] * NBUF
            + [pltpu.SemaphoreType.DMA] * (2 * NBUF)
        ),
    )(h, ei)


# speedup vs baseline: 1.0066x; 1.0066x over previous
import functools

import jax
import jax.numpy as jnp
from jax import lax
from jax.experimental import pallas as pl
from jax.experimental.pallas import tpu as pltpu
from jax.experimental.pallas import tpu_sc as plsc

NC, NS = 2, 16
NW = NC * NS
NBUF = 5


def _gather_body(h_hbm, ei_hbm, out_hbm, h_sp, idx_v, *bufs_and_sems,
                 ntask, ec, e_per_w, rows_per_tile):
    bufs = bufs_and_sems[:NBUF]
    sgs = bufs_and_sems[NBUF:2 * NBUF]
    sws = bufs_and_sems[2 * NBUF:3 * NBUF]
    sstage = bufs_and_sems[3 * NBUF:]
    cid = lax.axis_index("c")
    sid = lax.axis_index("s")
    wid = cid * NS + sid
    e0 = wid * e_per_w
    d = h_sp.shape[1]
    n_edges = ei_hbm.shape[0] // 2
    n_rows = h_sp.shape[0]
    off = pl.multiple_of(jnp.minimum(sid * rows_per_tile, n_rows - rows_per_tile), 8)
    cp_h = pltpu.make_async_copy(h_hbm.at[pl.ds(off, rows_per_tile)],
                                 h_sp.at[pl.ds(off, rows_per_tile)], sstage[0])
    cp_s = pltpu.make_async_copy(ei_hbm.at[pl.ds(e0, e_per_w)],
                                 idx_v.at[pl.ds(0, e_per_w)], sstage[1])
    cp_d = pltpu.make_async_copy(ei_hbm.at[pl.ds(n_edges + e0, e_per_w)],
                                 idx_v.at[pl.ds(e_per_w, e_per_w)], sstage[2])
    cp_h.start()
    cp_s.start()
    cp_d.start()
    cp_s.wait()
    cp_d.wait()

    def _idx(k):
        return pl.multiple_of((k % 2) * e_per_w + (k // 2) * ec, 8)

    def gather(k, i):
        return pltpu.make_async_copy(h_sp.at[idx_v.at[pl.ds(_idx(k), ec)]],
                                     bufs[i], sgs[i])

    def gather_hbm(k, i):
        return pltpu.make_async_copy(h_hbm.at[idx_v.at[pl.ds(_idx(k), ec)]],
                                     bufs[i], sgs[i])

    def writeback(k, i):
        rows = pl.ds(e0 + (k // 2) * ec, ec)
        col = pl.multiple_of((k % 2) * d, d)
        return pltpu.make_async_copy(bufs[i], out_hbm.at[rows, pl.ds(col, d)],
                                     sws[i])

    def slot(k, i):
        gather(k, i).wait()
        writeback(k, i).start()

        @pl.when(k > 0)
        def _():
            writeback(k - 1, (i - 1) % NBUF).wait()

        @pl.when(k + NBUF - 2 < ntask)
        def _():
            gather(k + NBUF - 2, (i + NBUF - 2) % NBUF).start()

    def group(t, carry):
        k0 = NBUF * t
        for i in range(NBUF):
            slot(k0 + i, i)
        return carry

    for i in range(3):
        gather_hbm(i, i).start()
    gather_hbm(0, 0).wait()
    writeback(0, 0).start()
    gather_hbm(3, 3).start()
    gather_hbm(1, 1).wait()
    writeback(1, 1).start()
    writeback(0, 0).wait()
    gather_hbm(4, 4).start()
    gather_hbm(2, 2).wait()
    writeback(2, 2).start()
    writeback(1, 1).wait()
    cp_h.wait()
    plsc.subcore_barrier()
    gather(5, 0).start()
    gather_hbm(3, 3).wait()
    writeback(3, 3).start()
    writeback(2, 2).wait()
    gather(6, 1).start()
    gather_hbm(4, 4).wait()
    writeback(4, 4).start()
    writeback(3, 3).wait()
    gather(7, 2).start()

    def group_tail(t, carry):
        k0 = NBUF + NBUF * t
        for i in range(NBUF):
            slot(k0 + i, i)
        return carry

    lax.fori_loop(0, ntask // NBUF - 1, group_tail, 0)
    writeback(ntask - 1, (ntask - 1) % NBUF).wait()


def kernel(h, edge_index):
    n, d = h.shape
    e = edge_index.shape[1]
    e_per_w = e // NW
    ec = 40
    ntask = 2 * (e_per_w // ec)
    assert ntask % NBUF == 0

    ei = edge_index.astype(jnp.int32).reshape(-1)
    rpt = ((n + NS - 1) // NS + 7) // 8 * 8
    body = functools.partial(_gather_body, ntask=ntask, ec=ec,
                             e_per_w=e_per_w, rows_per_tile=rpt)
    return pl.kernel(
        body,
        out_type=jax.ShapeDtypeStruct((e, 2 * d), jnp.float32),
        mesh=plsc.VectorSubcoreMesh(core_axis_name="c", subcore_axis_name="s"),
        scratch_types=(
            [pltpu.VMEM_SHARED((n, d), jnp.float32),
             pltpu.VMEM((2 * e_per_w,), jnp.int32)]
            + [pltpu.VMEM((ec, d), jnp.float32)] * NBUF
            + [pltpu.SemaphoreType.DMA] * (2 * NBUF + 3)
        ),
    )(h, ei)
